# Initial kernel scaffold; baseline (speedup 1.0000x reference)
import jax
import jax.numpy as jnp
from jax.experimental import pallas as pl


def kernel(gene, disease, chromosome, variant, ref_base, mut_base, position, zygosity, allele_freq, aa_orig_props, aa_mut_props, gene_table, disease_table, chromosome_table, variant_table, base_table, fc1_w, fc1_b, fc2_w, fc2_b, fc3_w, fc3_b):
    raise NotImplementedError("write your pallas kernel here")


# speedup vs baseline: 1.1598x; 1.1598x over previous
import jax
import jax.numpy as jnp
from jax import lax
from jax.experimental import pallas as pl
from jax.experimental.pallas import tpu as pltpu
from jax.experimental.pallas import tpu_sc as plsc

B = 16384
NC = 2
NS = 16
NW = NC * NS
BPW = B // NW
CHUNK = 128
NCH = BPW // CHUNK
BLK = 512


def _sc_gather_body(gene_i, dis_i, var_i, gene_t, dis_t, var_t,
                    out_g, out_d, out_v, idxv, rows, rows16, sem):
    wid = lax.axis_index("s") * NC + lax.axis_index("c")
    base = wid * BPW

    pltpu.sync_copy(gene_i.at[wid], idxv)
    cps = [pltpu.async_copy(gene_t.at[idxv.at[j]],
                            rows.at[pl.ds(j * CHUNK, CHUNK)], sem)
           for j in range(NCH)]
    for cp in cps:
        cp.wait()
    pltpu.sync_copy(rows, out_g.at[pl.ds(base, BPW)])

    pltpu.sync_copy(dis_i.at[wid], idxv)
    cps = [pltpu.async_copy(dis_t.at[idxv.at[j]],
                            rows.at[pl.ds(j * CHUNK, CHUNK)], sem)
           for j in range(NCH)]
    for cp in cps:
        cp.wait()
    pltpu.sync_copy(rows, out_d.at[pl.ds(base, BPW)])

    pltpu.sync_copy(var_i.at[wid], idxv)
    cps = [pltpu.async_copy(var_t.at[idxv.at[j]],
                            rows16.at[pl.ds(j * CHUNK, CHUNK)], sem)
           for j in range(NCH)]
    for cp in cps:
        cp.wait()
    pltpu.sync_copy(rows16, out_v.at[pl.ds(base, BPW)])


def _leaky(x):
    return jnp.where(x >= 0, x, 0.01 * x)


def _mlp_body(gene_r, dis_r, var_r, chrom_r, refb_r, mutb_r,
              pos_r, zyg_r, af_r, aao_r, aam_r,
              ct_r, bt_r,
              w1g_r, w1d_r, w1c_r, w1v_r, w1r_r, w1m_r, w1x_r,
              b1_r, w2_r, b2_r, w3_r, b3_r, out_r):
    f32 = jnp.float32
    ct_proj = jnp.dot(ct_r[...], w1c_r[...], preferred_element_type=f32)
    bt_proj_r = jnp.dot(bt_r[...], w1r_r[...], preferred_element_type=f32)
    bt_proj_m = jnp.dot(bt_r[...], w1m_r[...], preferred_element_type=f32)

    iota25 = lax.broadcasted_iota(jnp.int32, (BLK, 25), 1)
    iota5 = lax.broadcasted_iota(jnp.int32, (BLK, 5), 1)
    oh_c = (chrom_r[...] == iota25).astype(f32)
    oh_r = (refb_r[...] == iota5).astype(f32)
    oh_m = (mutb_r[...] == iota5).astype(f32)

    xd = jnp.concatenate([pos_r[...], zyg_r[...], af_r[...],
                          aao_r[...], aam_r[...]], axis=1)

    h1 = jnp.dot(gene_r[...], w1g_r[...], preferred_element_type=f32)
    h1 += jnp.dot(dis_r[...], w1d_r[...], preferred_element_type=f32)
    h1 += jnp.dot(var_r[...], w1v_r[...], preferred_element_type=f32)
    h1 += jnp.dot(oh_c, ct_proj, preferred_element_type=f32)
    h1 += jnp.dot(oh_r, bt_proj_r, preferred_element_type=f32)
    h1 += jnp.dot(oh_m, bt_proj_m, preferred_element_type=f32)
    h1 += jnp.dot(xd, w1x_r[...], preferred_element_type=f32)
    h1 += b1_r[...]
    h1 = _leaky(h1)

    h2 = _leaky(jnp.dot(h1, w2_r[...], preferred_element_type=f32) + b2_r[...])
    h3 = jnp.dot(h2, w3_r[...], preferred_element_type=f32) + b3_r[...]
    out_r[...] = 1.0 / (1.0 + jnp.exp(-h3))


def kernel(gene, disease, chromosome, variant, ref_base, mut_base, position,
           zygosity, allele_freq, aa_orig_props, aa_mut_props,
           gene_table, disease_table, chromosome_table, variant_table,
           base_table, fc1_w, fc1_b, fc2_w, fc2_b, fc3_w, fc3_b):
    f32 = jnp.float32

    gi = gene.reshape(NW, NCH, CHUNK)
    di = disease.reshape(NW, NCH, CHUNK)
    vi = variant.reshape(NW, NCH, CHUNK)
    mesh = plsc.VectorSubcoreMesh(core_axis_name="c", subcore_axis_name="s")
    sc = pl.kernel(
        _sc_gather_body,
        out_type=(jax.ShapeDtypeStruct((B, 128), f32),
                  jax.ShapeDtypeStruct((B, 128), f32),
                  jax.ShapeDtypeStruct((B, 16), f32)),
        mesh=mesh,
        scratch_types=(pltpu.VMEM((NCH, CHUNK), jnp.int32),
                       pltpu.VMEM((BPW, 128), f32),
                       pltpu.VMEM((BPW, 16), f32),
                       pltpu.SemaphoreType.DMA),
        compiler_params=pltpu.CompilerParams(use_tc_tiling_on_sc=False),
    )
    gene_rows, dis_rows, var_rows = sc(gi, di, vi, gene_table, disease_table,
                                       variant_table)

    w1 = fc1_w.T
    w1g, w1d, w1c = w1[0:128], w1[128:256], w1[256:320]
    w1v, w1r, w1m, w1x = w1[320:336], w1[336:352], w1[352:368], w1[368:419]
    b1 = fc1_b.reshape(1, 128)
    w2 = fc2_w.T
    b2 = fc2_b.reshape(1, 64)
    w3 = fc3_w.T
    b3 = fc3_b.reshape(1, 1)
    chrom2 = chromosome.reshape(B, 1)
    ref2 = ref_base.reshape(B, 1)
    mut2 = mut_base.reshape(B, 1)
    pos2 = position.reshape(B, 1)
    zyg2 = zygosity.reshape(B, 1)

    grid = (B // BLK,)

    def row(d):
        return pl.BlockSpec((BLK, d), lambda i: (i, 0))

    def full(shape):
        return pl.BlockSpec(shape, lambda i: (0,) * len(shape))

    out = pl.pallas_call(
        _mlp_body,
        grid=grid,
        in_specs=[row(128), row(128), row(16), row(1), row(1), row(1),
                  row(1), row(1), row(9), row(20), row(20),
                  full((25, 64)), full((5, 16)),
                  full((128, 128)), full((128, 128)), full((64, 128)),
                  full((16, 128)), full((16, 128)), full((16, 128)),
                  full((51, 128)),
                  full((1, 128)), full((128, 64)), full((1, 64)),
                  full((64, 1)), full((1, 1))],
        out_specs=pl.BlockSpec((BLK, 1), lambda i: (i, 0)),
        out_shape=jax.ShapeDtypeStruct((B, 1), f32),
    )(gene_rows, dis_rows, var_rows, chrom2, ref2, mut2,
      pos2, zyg2, allele_freq, aa_orig_props, aa_mut_props,
      chromosome_table, base_table,
      w1g, w1d, w1c, w1v, w1r, w1m, w1x,
      b1, w2, b2, w3, b3)
    return out
